# baseline (speedup 1.0000x reference)
import functools

import jax
import jax.numpy as jnp
from jax.experimental import pallas as pl
from jax.experimental.pallas import tpu as pltpu

_B = 1600
_S = 160
_W = 32
_NEG = -3.0e38


def _body(nb, lo_hi_ref, feats_ref, aug_t_ref, ids_col_ref, ids_row_ref,
          gamma_ref, beta_ref, w_ref, b_ref, feats_out_ref, aux_out_ref):
    i = pl.program_id(0)

    @pl.when(i == 0)
    def _init():
        feats_out_ref[...] = jnp.full_like(feats_out_ref, _NEG)
        aux_out_ref[...] = jnp.zeros_like(aux_out_ref)

    x = feats_ref[...]
    mean = jnp.mean(x, axis=1, keepdims=True)
    r = x - mean
    var = jnp.mean(r * r, axis=1, keepdims=True)
    normed = r * jax.lax.rsqrt(var + 1e-5) * gamma_ref[...] + beta_ref[...]
    lin = jnp.dot(normed.astype(jnp.bfloat16), w_ref[...],
                  preferred_element_type=jnp.float32) + b_ref[...]

    bits_col = ids_col_ref[...]
    ids_row = ids_row_ref[0]
    bsz = lin.shape[0]

    x = jnp.ones((1, bsz), jnp.int32)
    k = 1
    while k < bsz:
        sh_x = jnp.concatenate(
            [jnp.zeros((1, k), jnp.int32), x[:, : bsz - k]], axis=1)
        sh_i = jnp.concatenate(
            [jnp.full((1, k), -1, jnp.int32), ids_row[:, : bsz - k]], axis=1)
        x = x + jnp.where(ids_row == sh_i, sh_x, 0)
        k *= 2
    r_row = x - 1
    nphases = jnp.max(r_row) // _W + 1

    v = lin.astype(jnp.bfloat16)
    k, kk = 1, 0
    while k < _W:
        sh_v = jnp.concatenate(
            [v[k:], jnp.full((k, v.shape[1]), _NEG, jnp.bfloat16)], axis=0)
        same = (bits_col & (1 << kk)) != 0
        v = jnp.where(same, jnp.maximum(v, sh_v), v)
        k, kk = k * 2, kk + 1

    aug_t = aug_t_ref[0]
    aug_hi = aug_t[:4]
    aug_lo = aug_t[4:]

    lo = lo_hi_ref[0, 0, 0]
    hi = lo_hi_ref[0, 0, 1]
    t_lo = lo // _S
    t_hi = hi // _S
    n_tiles = feats_out_ref.shape[0]

    d_iota = jax.lax.broadcasted_iota(jnp.int32, (_S, bsz), 0)
    d_iota2 = jax.lax.broadcasted_iota(jnp.int32, (2 * _S, bsz), 0)

    tp = jnp.minimum(t_lo, n_tiles - 2)
    base0 = tp * _S
    sel2 = ((ids_row - base0) == d_iota2).astype(jnp.bfloat16)
    sums2 = (
        jax.lax.dot_general(
            aug_hi, sel2, (((1,), (1,)), ((), ())),
            preferred_element_type=jnp.float32)
        + jax.lax.dot_general(
            aug_lo, sel2, (((1,), (1,)), ((), ())),
            preferred_element_type=jnp.float32))
    aux_out_ref[pl.ds(tp, 1)] += sums2[:, :_S][None]
    aux_out_ref[pl.ds(tp + 1, 1)] += sums2[:, _S:][None]
    cnt2 = jnp.transpose(sums2[3:4, :])

    zero_bf = jnp.zeros((), jnp.bfloat16)

    def pair_phase(c, carry):
        p = jnp.where(r_row == c * _W, sel2, zero_bf)
        placed = jax.lax.dot_general(
            p, v, (((1,), (0,)), ((), ())),
            preferred_element_type=jnp.float32)
        has = cnt2 > (c * _W)
        f_pair = feats_out_ref[pl.ds(tp, 2)]
        feats_out_ref[pl.ds(tp, 2)] = jnp.where(
            has.reshape(2, _S, 1),
            jnp.maximum(f_pair, placed.reshape(2, _S, v.shape[1])), f_pair)
        return carry

    pair_phase(0, 0)
    jax.lax.fori_loop(1, nphases, pair_phase, 0)

    def do_tile(t):
        base = t * _S
        sel_f = ((ids_row - base) == d_iota).astype(jnp.bfloat16)
        sums_t = (
            jax.lax.dot_general(
                aug_hi, sel_f, (((1,), (1,)), ((), ())),
                preferred_element_type=jnp.float32)
            + jax.lax.dot_general(
                aug_lo, sel_f, (((1,), (1,)), ((), ())),
                preferred_element_type=jnp.float32))
        aux_out_ref[pl.ds(t, 1)] += sums_t[None]
        cnt_t = jnp.transpose(sums_t[3:4, :])

        def tile_phase(c, carry):
            p = jnp.where(r_row == c * _W, sel_f, zero_bf)
            placed = jax.lax.dot_general(
                p, v, (((1,), (0,)), ((), ())),
                preferred_element_type=jnp.float32)
            has = cnt_t > (c * _W)
            f_tile = feats_out_ref[pl.ds(t, 1)]
            feats_out_ref[pl.ds(t, 1)] = jnp.where(
                has[None], jnp.maximum(f_tile, placed[None]), f_tile)
            return carry
        tile_phase(0, 0)
        jax.lax.fori_loop(1, nphases, tile_phase, 0)

    def tbody(t, carry):
        do_tile(t)
        return carry
    jax.lax.fori_loop(tp + 2, t_hi + 1, tbody, 0)

    @pl.when(i == nb - 1)
    def _fin():
        f = feats_out_ref[...]
        feats_out_ref[...] = jnp.where(f > -1.0e37, f, 0.0)
        a = aux_out_ref[...]
        aux_out_ref[...] = a / jnp.clip(a[:, 3:4, :], 1.0, None)


def kernel(feats, coords, segment_ids, ln_gamma, ln_beta, W, b):
    n, d_in = feats.shape
    d_out = W.shape[1]
    num_seg = 40000
    assert n % _B == 0 and num_seg % _S == 0
    nb = n // _B

    pos = jnp.arange(n, dtype=jnp.int32) % _B
    bits = jnp.zeros((n,), jnp.int32)
    kk = 0
    step = 1
    while step < _W:
        same = jnp.concatenate(
            [segment_ids[step:] == segment_ids[:-step],
             jnp.zeros((step,), bool)])
        same = same & (pos + step < _B)
        bits = bits | (same.astype(jnp.int32) << kk)
        kk, step = kk + 1, step * 2
    ids_col = bits.reshape(n, 1)
    ids_row = segment_ids.reshape(nb, 1, _B)
    ids2d = segment_ids.reshape(nb, _B)
    lo_hi = jnp.stack([ids2d[:, 0], ids2d[:, -1]], axis=1).reshape(nb, 1, 2)
    aug = jnp.concatenate(
        [coords, jnp.ones((n, 1), jnp.float32)], axis=1)
    aug_hi = aug.astype(jnp.bfloat16)
    aug_lo = (aug - aug_hi.astype(jnp.float32)).astype(jnp.bfloat16)
    aug_t = jnp.concatenate(
        [aug_hi.reshape(nb, _B, 4), aug_lo.reshape(nb, _B, 4)],
        axis=2).transpose(0, 2, 1)

    grid = (nb,)
    out = pl.pallas_call(
        functools.partial(_body, nb),
        grid=grid,
        in_specs=[
            pl.BlockSpec((1, 1, 2), lambda i: (i, 0, 0),
                         memory_space=pltpu.SMEM),
            pl.BlockSpec((_B, d_in), lambda i: (i, 0)),
            pl.BlockSpec((1, 8, _B), lambda i: (i, 0, 0)),
            pl.BlockSpec((_B, 1), lambda i: (i, 0)),
            pl.BlockSpec((1, 1, _B), lambda i: (i, 0, 0)),
            pl.BlockSpec((1, d_in), lambda i: (0, 0)),
            pl.BlockSpec((1, d_in), lambda i: (0, 0)),
            pl.BlockSpec((d_in, d_out), lambda i: (0, 0)),
            pl.BlockSpec((1, d_out), lambda i: (0, 0)),
        ],
        out_specs=[
            pl.BlockSpec((num_seg // _S, _S, d_out), lambda i: (0, 0, 0)),
            pl.BlockSpec((num_seg // _S, 4, _S), lambda i: (0, 0, 0)),
        ],
        out_shape=[
            jax.ShapeDtypeStruct((num_seg // _S, _S, d_out), jnp.float32),
            jax.ShapeDtypeStruct((num_seg // _S, 4, _S), jnp.float32),
        ],
        compiler_params=pltpu.CompilerParams(
            dimension_semantics=("arbitrary",)),
    )(lo_hi, feats, aug_t, ids_col, ids_row,
      ln_gamma.reshape(1, d_in), ln_beta.reshape(1, d_in),
      W.astype(jnp.bfloat16), b.reshape(1, d_out))
    feats_down, aux = out
    coords_down = aux.transpose(1, 0, 2).reshape(4, num_seg)[:3, :].T
    return feats_down.reshape(num_seg, d_out), coords_down


# speedup vs baseline: 1.0816x; 1.0816x over previous
import functools

import jax
import jax.numpy as jnp
from jax.experimental import pallas as pl
from jax.experimental.pallas import tpu as pltpu

_B = 1600
_S = 160
_W = 32
_NEG = -3.0e38


def _body(nb, lo_hi_ref, feats_ref, aug_t_ref, ids_col_ref, ids_row_ref,
          gamma_ref, beta_ref, w_ref, b_ref, feats_out_ref, aux_out_ref):
    i = pl.program_id(0)

    @pl.when(i == 0)
    def _init():
        feats_out_ref[...] = jnp.full_like(feats_out_ref, _NEG)
        aux_out_ref[...] = jnp.zeros_like(aux_out_ref)

    x = feats_ref[...]
    mean = jnp.mean(x, axis=1, keepdims=True)
    r = x - mean
    var = jnp.mean(r * r, axis=1, keepdims=True)
    normed = r * jax.lax.rsqrt(var + 1e-5) * gamma_ref[...] + beta_ref[...]
    lin = jnp.dot(normed.astype(jnp.bfloat16), w_ref[...],
                  preferred_element_type=jnp.float32) + b_ref[...]

    bits_col = ids_col_ref[...]
    ids_row = ids_row_ref[0]
    bsz = lin.shape[0]

    x = jnp.ones((1, bsz), jnp.int32)
    k = 1
    while k < bsz:
        sh_x = jnp.concatenate(
            [jnp.zeros((1, k), jnp.int32), x[:, : bsz - k]], axis=1)
        sh_i = jnp.concatenate(
            [jnp.full((1, k), -1, jnp.int32), ids_row[:, : bsz - k]], axis=1)
        x = x + jnp.where(ids_row == sh_i, sh_x, 0)
        k *= 2
    r_row = x - 1
    nphases = jnp.max(r_row) // _W + 1

    v = lin.astype(jnp.bfloat16)
    k, kk = 1, 0
    while k < _W:
        sh_v = jnp.concatenate(
            [v[k:], jnp.full((k, v.shape[1]), _NEG, jnp.bfloat16)], axis=0)
        same = (bits_col & (1 << kk)) != 0
        v = jnp.where(same, jnp.maximum(v, sh_v), v)
        k, kk = k * 2, kk + 1

    aug_t = aug_t_ref[0]

    lo = lo_hi_ref[0, 0, 0]
    hi = lo_hi_ref[0, 0, 1]
    t_lo = lo // _S
    t_hi = hi // _S
    n_tiles = feats_out_ref.shape[0]

    d_iota = jax.lax.broadcasted_iota(jnp.int32, (_S, bsz), 0)
    d_iota2 = jax.lax.broadcasted_iota(jnp.int32, (2 * _S, bsz), 0)

    tp = jnp.minimum(t_lo, n_tiles - 2)
    base0 = tp * _S
    sel2 = ((ids_row - base0) == d_iota2).astype(jnp.float32)
    sums2 = jax.lax.dot_general(
        aug_t, sel2, (((1,), (1,)), ((), ())),
        preferred_element_type=jnp.float32)
    aux_out_ref[pl.ds(tp, 1)] += sums2[:, :_S][None]
    aux_out_ref[pl.ds(tp + 1, 1)] += sums2[:, _S:][None]
    cnt2 = jnp.transpose(sums2[3:4, :])

    def pair_phase(c, carry):
        p = jnp.where(r_row == c * _W, sel2, 0.0)
        placed = jax.lax.dot_general(
            p.astype(jnp.bfloat16), v, (((1,), (0,)), ((), ())),
            preferred_element_type=jnp.float32)
        has = cnt2 > (c * _W)
        f_pair = feats_out_ref[pl.ds(tp, 2)]
        feats_out_ref[pl.ds(tp, 2)] = jnp.where(
            has.reshape(2, _S, 1),
            jnp.maximum(f_pair, placed.reshape(2, _S, v.shape[1])), f_pair)
        return carry

    pair_phase(0, 0)
    jax.lax.fori_loop(1, nphases, pair_phase, 0)

    def do_tile(t):
        base = t * _S
        sel_f = ((ids_row - base) == d_iota).astype(jnp.float32)
        sums_t = jax.lax.dot_general(
            aug_t, sel_f, (((1,), (1,)), ((), ())),
            preferred_element_type=jnp.float32)
        aux_out_ref[pl.ds(t, 1)] += sums_t[None]
        cnt_t = jnp.transpose(sums_t[3:4, :])

        def tile_phase(c, carry):
            p = jnp.where(r_row == c * _W, sel_f, 0.0)
            placed = jax.lax.dot_general(
                p.astype(jnp.bfloat16), v, (((1,), (0,)), ((), ())),
                preferred_element_type=jnp.float32)
            has = cnt_t > (c * _W)
            f_tile = feats_out_ref[pl.ds(t, 1)]
            feats_out_ref[pl.ds(t, 1)] = jnp.where(
                has[None], jnp.maximum(f_tile, placed[None]), f_tile)
            return carry
        tile_phase(0, 0)
        jax.lax.fori_loop(1, nphases, tile_phase, 0)

    def tbody(t, carry):
        do_tile(t)
        return carry
    jax.lax.fori_loop(tp + 2, t_hi + 1, tbody, 0)

    @pl.when(i == nb - 1)
    def _fin():
        f = feats_out_ref[...]
        feats_out_ref[...] = jnp.where(f > -1.0e37, f, 0.0)
        a = aux_out_ref[...]
        aux_out_ref[...] = a / jnp.clip(a[:, 3:4, :], 1.0, None)


def kernel(feats, coords, segment_ids, ln_gamma, ln_beta, W, b):
    n, d_in = feats.shape
    d_out = W.shape[1]
    num_seg = 40000
    assert n % _B == 0 and num_seg % _S == 0
    nb = n // _B

    pos = jnp.arange(n, dtype=jnp.int32) % _B
    bits = jnp.zeros((n,), jnp.int32)
    kk = 0
    step = 1
    while step < _W:
        same = jnp.concatenate(
            [segment_ids[step:] == segment_ids[:-step],
             jnp.zeros((step,), bool)])
        same = same & (pos + step < _B)
        bits = bits | (same.astype(jnp.int32) << kk)
        kk, step = kk + 1, step * 2
    ids_col = bits.reshape(n, 1)
    ids_row = segment_ids.reshape(nb, 1, _B)
    ids2d = segment_ids.reshape(nb, _B)
    lo_hi = jnp.stack([ids2d[:, 0], ids2d[:, -1]], axis=1).reshape(nb, 1, 2)
    aug_t = jnp.concatenate(
        [coords, jnp.ones((n, 1), jnp.float32)],
        axis=1).reshape(nb, _B, 4).transpose(0, 2, 1)

    grid = (nb,)
    out = pl.pallas_call(
        functools.partial(_body, nb),
        grid=grid,
        in_specs=[
            pl.BlockSpec((1, 1, 2), lambda i: (i, 0, 0),
                         memory_space=pltpu.SMEM),
            pl.BlockSpec((_B, d_in), lambda i: (i, 0)),
            pl.BlockSpec((1, 4, _B), lambda i: (i, 0, 0)),
            pl.BlockSpec((_B, 1), lambda i: (i, 0)),
            pl.BlockSpec((1, 1, _B), lambda i: (i, 0, 0)),
            pl.BlockSpec((1, d_in), lambda i: (0, 0)),
            pl.BlockSpec((1, d_in), lambda i: (0, 0)),
            pl.BlockSpec((d_in, d_out), lambda i: (0, 0)),
            pl.BlockSpec((1, d_out), lambda i: (0, 0)),
        ],
        out_specs=[
            pl.BlockSpec((num_seg // _S, _S, d_out), lambda i: (0, 0, 0)),
            pl.BlockSpec((num_seg // _S, 4, _S), lambda i: (0, 0, 0)),
        ],
        out_shape=[
            jax.ShapeDtypeStruct((num_seg // _S, _S, d_out), jnp.float32),
            jax.ShapeDtypeStruct((num_seg // _S, 4, _S), jnp.float32),
        ],
        compiler_params=pltpu.CompilerParams(
            dimension_semantics=("arbitrary",)),
    )(lo_hi, feats, aug_t, ids_col, ids_row,
      ln_gamma.reshape(1, d_in), ln_beta.reshape(1, d_in),
      W.astype(jnp.bfloat16), b.reshape(1, d_out))
    feats_down, aux = out
    coords_down = aux.transpose(1, 0, 2).reshape(4, num_seg)[:3, :].T
    return feats_down.reshape(num_seg, d_out), coords_down
